# baseline (speedup 1.0000x reference)
import functools

import jax
import jax.numpy as jnp
from jax import lax
from jax.experimental import pallas as pl
from jax.experimental.pallas import tpu as pltpu
from jax.experimental.pallas import tpu_sc as plsc


def _stage1_body(x1_ref, x2t_ref,
                 i0_ref, i1_ref, i2_ref, w0_ref, w1_ref, w2_ref,
                 *, blk, s):
    bi = pl.program_id(0)

    x1 = x1_ref[0]
    x2t = x2t_ref[0]
    cross = jnp.dot(x1, x2t, preferred_element_type=jnp.float32)
    n1 = jnp.sum(x1 * x1, axis=1, keepdims=True)
    n2 = jnp.sum(x2t * x2t, axis=0, keepdims=True)
    d = -2.0 * cross
    d = d + n1
    d = d + n2
    d = jnp.maximum(d, jnp.float32(0.0001))

    ch = 128
    nch = s // ch
    m1 = d[:, 0:ch]
    inf = jnp.full((blk, ch), jnp.inf, jnp.float32)
    m2 = inf
    m3 = inf
    for c in range(1, nch):
        cv = d[:, c * ch:(c + 1) * ch]
        nm1 = jnp.minimum(m1, cv)
        pu = jnp.maximum(m1, cv)
        nm2 = jnp.minimum(m2, pu)
        pu2 = jnp.maximum(m2, pu)
        m3 = jnp.minimum(m3, pu2)
        m1, m2 = nm1, nm2
    off = 1
    while off < ch:
        r1 = pltpu.roll(m1, ch - off, 1)
        r2 = pltpu.roll(m2, ch - off, 1)
        r3 = pltpu.roll(m3, ch - off, 1)
        p = jnp.maximum(m1, r1)
        q = jnp.minimum(m2, r2)
        c1 = jnp.minimum(m1, r1)
        c2 = jnp.minimum(p, q)
        c3 = jnp.minimum(jnp.maximum(p, q), jnp.minimum(m3, r3))
        m1, m2, m3 = c1, c2, c3
        off *= 2
    v1 = m1[:, 0:1]
    v2 = m2[:, 0:1]
    v3 = m3[:, 0:1]

    iota = jax.lax.broadcasted_iota(jnp.int32, (blk, s), 1)
    big = jnp.int32(s)
    i1 = jnp.min(jnp.where(d == v1, iota, big), axis=1, keepdims=True)
    i2 = jnp.min(jnp.where((d == v2) & (iota != i1), iota, big),
                 axis=1, keepdims=True)
    i3 = jnp.min(jnp.where((d == v3) & (iota != i1) & (iota != i2),
                           iota, big), axis=1, keepdims=True)

    r1w = 1.0 / (v1 + jnp.float32(0.0001))
    r2w = 1.0 / (v2 + jnp.float32(0.0001))
    r3w = 1.0 / (v3 + jnp.float32(0.0001))
    scale = 1.0 / (r1w + r2w + r3w + jnp.float32(0.0001))

    base = bi * jnp.int32(s)
    i0_ref[0] = i1 + base
    i1_ref[0] = i2 + base
    i2_ref[0] = i3 + base
    w0_ref[0] = r1w * scale
    w1_ref[0] = r2w * scale
    w2_ref[0] = r3w * scale


def _sc_gather(idx0, idx1, idx2, table, *, bn, c2):
    info = plsc.get_sparse_core_info()
    nw = info.num_cores * info.num_subcores
    per_w = bn // nw
    chunk = 128
    nch = per_w // chunk
    mesh = plsc.VectorSubcoreMesh(core_axis_name="c", subcore_axis_name="s")

    @functools.partial(
        pl.kernel, mesh=mesh,
        out_type=[jax.ShapeDtypeStruct((bn, c2), jnp.float32)
                  for _ in range(3)],
        scratch_types=[
            pltpu.VMEM((per_w,), jnp.int32),
            pltpu.VMEM((per_w,), jnp.int32),
            pltpu.VMEM((per_w,), jnp.int32),
            pltpu.VMEM((chunk, c2), jnp.float32),
            pltpu.VMEM((chunk, c2), jnp.float32),
            pltpu.SemaphoreType.DMA,
            pltpu.SemaphoreType.DMA,
        ],
    )
    def k(i0_hbm, i1_hbm, i2_hbm, tab_hbm, o0_hbm, o1_hbm, o2_hbm,
          iv0, iv1, iv2, buf0, buf1, sem0, sem1):
        wid = lax.axis_index("s") * info.num_cores + lax.axis_index("c")
        base = wid * per_w
        pltpu.sync_copy(i0_hbm.at[pl.ds(base, per_w)], iv0)
        pltpu.sync_copy(i1_hbm.at[pl.ds(base, per_w)], iv1)
        pltpu.sync_copy(i2_hbm.at[pl.ds(base, per_w)], iv2)

        ivs = (iv0, iv1, iv2)
        outs = (o0_hbm, o1_hbm, o2_hbm)
        bufs = (buf0, buf1)
        sems = (sem0, sem1)
        pairs = [(k_, c_) for k_ in range(3) for c_ in range(nch)]

        def start(j):
            k_, c_ = pairs[j]
            return pltpu.async_copy(
                tab_hbm.at[ivs[k_].at[pl.ds(c_ * chunk, chunk)]],
                bufs[j % 2], sems[j % 2])

        cps = [start(0), None]
        for j in range(len(pairs)):
            if j + 1 < len(pairs):
                cps[(j + 1) % 2] = start(j + 1)
            cps[j % 2].wait()
            k_, c_ = pairs[j]
            pltpu.sync_copy(bufs[j % 2],
                            outs[k_].at[pl.ds(base + c_ * chunk, chunk)])

    return k(idx0, idx1, idx2, table)


def _stage2_body(p1_ref, g0_ref, g1_ref, g2_ref, w0_ref, w1_ref, w2_ref,
                 w1a_ref, w1b_ref, b1_ref, h1_ref, stats_ref):
    i = pl.program_id(0)
    interp = (g0_ref[...] * w0_ref[...] + g1_ref[...] * w1_ref[...]
              + g2_ref[...] * w2_ref[...])
    h1 = (jnp.dot(p1_ref[...], w1a_ref[...], preferred_element_type=jnp.float32)
          + jnp.dot(interp, w1b_ref[...], preferred_element_type=jnp.float32)
          + b1_ref[...])
    h1_ref[...] = h1

    @pl.when(i == 0)
    def _init():
        stats_ref[...] = jnp.zeros_like(stats_ref)

    stats_ref[0:1, :] += jnp.sum(h1, axis=0, keepdims=True)
    stats_ref[1:2, :] += jnp.sum(h1 * h1, axis=0, keepdims=True)


def _stage3_body(h1_ref, sc1_ref, sh1_ref, w2_ref, b2_ref,
                 h2_ref, stats_ref):
    i = pl.program_id(0)
    h = h1_ref[...]
    h = jnp.maximum(h * sc1_ref[...] + sh1_ref[...], jnp.float32(0.0))
    h2 = jnp.dot(h, w2_ref[...], preferred_element_type=jnp.float32) + b2_ref[...]
    h2_ref[...] = h2

    @pl.when(i == 0)
    def _init():
        stats_ref[...] = jnp.zeros_like(stats_ref)

    stats_ref[0:1, :] += jnp.sum(h2, axis=0, keepdims=True)
    stats_ref[1:2, :] += jnp.sum(h2 * h2, axis=0, keepdims=True)


def _stage4_body(h2_ref, sc2_ref, sh2_ref, out_ref):
    out_ref[...] = jnp.maximum(
        h2_ref[...] * sc2_ref[...] + sh2_ref[...], jnp.float32(0.0))


def kernel(xyz1, xyz2, points1, points2, W1, b1, g1, be1, W2, b2, g2, be2):
    B, N, _ = xyz1.shape
    S = xyz2.shape[1]
    C1 = points1.shape[2]
    C2 = points2.shape[2]
    C = W1.shape[1]
    BN = B * N

    blk1 = 256 if N % 256 == 0 else N
    blk2 = 512 if BN % 512 == 0 else BN

    xyz1p = jnp.pad(xyz1, ((0, 0), (0, 0), (0, 5)))
    xyz2t = jnp.transpose(jnp.pad(xyz2, ((0, 0), (0, 0), (0, 5))),
                          (0, 2, 1))
    W1a = W1[:C1]
    W1b = W1[C1:]
    b1r = b1.reshape(1, C)
    b2r = b2.reshape(1, C)

    nblk = N // blk1
    idx_w = pl.pallas_call(
        functools.partial(_stage1_body, blk=blk1, s=S),
        grid=(B, nblk),
        in_specs=[
            pl.BlockSpec((1, blk1, 8), lambda b, n: (b, n, 0)),
            pl.BlockSpec((1, 8, S), lambda b, n: (b, 0, 0)),
        ],
        out_specs=[pl.BlockSpec((1, blk1, 1), lambda b, n: (b, n, 0))
                   for _ in range(6)],
        out_shape=[jax.ShapeDtypeStruct((B, N, 1), jnp.int32)
                   for _ in range(3)]
                  + [jax.ShapeDtypeStruct((B, N, 1), jnp.float32)
                     for _ in range(3)],
        compiler_params=pltpu.CompilerParams(
            dimension_semantics=("arbitrary", "arbitrary")),
    )(xyz1p, xyz2t)
    i0, i1, i2, w0, w1, w2 = idx_w

    table = points2.reshape(B * S, C2)
    ga, gb, gc = _sc_gather(i0.reshape(BN), i1.reshape(BN), i2.reshape(BN),
                            table, bn=BN, c2=C2)

    p1f = points1.reshape(BN, C1)
    nblk2 = BN // blk2
    cspec = pl.BlockSpec((blk2, C2), lambda i: (i, 0))
    wspec = pl.BlockSpec((blk2, 1), lambda i: (i, 0))
    h1, stats1 = pl.pallas_call(
        _stage2_body,
        grid=(nblk2,),
        in_specs=[
            pl.BlockSpec((blk2, C1), lambda i: (i, 0)),
            cspec, cspec, cspec, wspec, wspec, wspec,
            pl.BlockSpec((C1, C), lambda i: (0, 0)),
            pl.BlockSpec((C2, C), lambda i: (0, 0)),
            pl.BlockSpec((1, C), lambda i: (0, 0)),
        ],
        out_specs=[
            pl.BlockSpec((blk2, C), lambda i: (i, 0)),
            pl.BlockSpec((8, C), lambda i: (0, 0)),
        ],
        out_shape=[
            jax.ShapeDtypeStruct((BN, C), jnp.float32),
            jax.ShapeDtypeStruct((8, C), jnp.float32),
        ],
        compiler_params=pltpu.CompilerParams(
            dimension_semantics=("arbitrary",)),
    )(p1f, ga, gb, gc, w0.reshape(BN, 1), w1.reshape(BN, 1),
      w2.reshape(BN, 1), W1a, W1b, b1r)

    cnt = jnp.float32(BN)
    mean1 = stats1[0:1] / cnt
    var1 = stats1[1:2] / cnt - mean1 * mean1
    sc1v = g1.reshape(1, C) / jnp.sqrt(var1 + 1e-5)
    sh1v = be1.reshape(1, C) - mean1 * sc1v

    h2, stats2 = pl.pallas_call(
        _stage3_body,
        grid=(nblk2,),
        in_specs=[
            pl.BlockSpec((blk2, C), lambda i: (i, 0)),
            pl.BlockSpec((1, C), lambda i: (0, 0)),
            pl.BlockSpec((1, C), lambda i: (0, 0)),
            pl.BlockSpec((C, C), lambda i: (0, 0)),
            pl.BlockSpec((1, C), lambda i: (0, 0)),
        ],
        out_specs=[
            pl.BlockSpec((blk2, C), lambda i: (i, 0)),
            pl.BlockSpec((8, C), lambda i: (0, 0)),
        ],
        out_shape=[
            jax.ShapeDtypeStruct((BN, C), jnp.float32),
            jax.ShapeDtypeStruct((8, C), jnp.float32),
        ],
        compiler_params=pltpu.CompilerParams(
            dimension_semantics=("arbitrary",)),
    )(h1, sc1v, sh1v, W2, b2r)

    mean2 = stats2[0:1] / cnt
    var2 = stats2[1:2] / cnt - mean2 * mean2
    sc2v = g2.reshape(1, C) / jnp.sqrt(var2 + 1e-5)
    sh2v = be2.reshape(1, C) - mean2 * sc2v

    out = pl.pallas_call(
        _stage4_body,
        grid=(nblk2,),
        in_specs=[
            pl.BlockSpec((blk2, C), lambda i: (i, 0)),
            pl.BlockSpec((1, C), lambda i: (0, 0)),
            pl.BlockSpec((1, C), lambda i: (0, 0)),
        ],
        out_specs=pl.BlockSpec((blk2, C), lambda i: (i, 0)),
        out_shape=jax.ShapeDtypeStruct((BN, C), jnp.float32),
    )(h2, sc2v, sh2v)

    return out.reshape(B, N, C)
